# Initial kernel scaffold; baseline (speedup 1.0000x reference)
import jax
import jax.numpy as jnp
from jax.experimental import pallas as pl


def kernel(pred_logits, pred_boxes, tgt_boxes, positive_map):
    raise NotImplementedError("write your pallas kernel here")


# speedup vs baseline: 3.9773x; 3.9773x over previous
import jax
import jax.numpy as jnp
from jax import lax
from jax.experimental import pallas as pl
from jax.experimental.pallas import tpu as pltpu

_COST_CLASS = 2.0
_COST_BBOX = 5.0
_COST_GIOU = 2.0
_ALPHA = 0.25

_BS, _Q, _C, _T, _P = 8, 900, 256, 100, 4
_BIG = 1e9


def _cost_body(logits_ref, boxes_ref, tboxT_ref, pmapT_ref, c_ref):
    logits = logits_ref[0]
    p = jax.nn.sigmoid(logits)
    one_m_p = 1.0 - p
    neg = (1.0 - _ALPHA) * (p * p) * (-jnp.log(one_m_p))
    pos = _ALPHA * (one_m_p * one_m_p) * (-jnp.log(p + 1e-8))
    diff = pos - neg

    cio = lax.broadcasted_iota(jnp.int32, (_C, _T), 0)
    gs = []
    for pp in range(_P):
        idx = pmapT_ref[0, pp : pp + 1, :]
        onehot = (cio == idx).astype(jnp.float32)
        gs.append(lax.dot_general(
            diff, onehot, (((1,), (0,)), ((), ())),
            preferred_element_type=jnp.float32,
            precision=lax.Precision.HIGHEST,
        ))
    cost_class = ((gs[0] + gs[2]) + (gs[1] + gs[3])) * (1.0 / _P)

    boxes = boxes_ref[0]
    bcx, bcy, bw, bh = (boxes[:, k : k + 1] for k in range(4))
    tb = tboxT_ref[0]
    tcx, tcy, tw, th = (tb[k : k + 1, :] for k in range(4))

    cost_bbox = ((jnp.abs(bcx - tcx) + jnp.abs(bcy - tcy))
                 + jnp.abs(bw - tw)) + jnp.abs(bh - th)

    bx0, by0 = bcx - 0.5 * bw, bcy - 0.5 * bh
    bx1, by1 = bcx + 0.5 * bw, bcy + 0.5 * bh
    tx0, ty0 = tcx - 0.5 * tw, tcy - 0.5 * th
    tx1, ty1 = tcx + 0.5 * tw, tcy + 0.5 * th

    area1 = (bx1 - bx0) * (by1 - by0)
    area2 = (tx1 - tx0) * (ty1 - ty0)
    ltx, lty = jnp.maximum(bx0, tx0), jnp.maximum(by0, ty0)
    rbx, rby = jnp.minimum(bx1, tx1), jnp.minimum(by1, ty1)
    whx, why = jnp.maximum(rbx - ltx, 0.0), jnp.maximum(rby - lty, 0.0)
    inter = whx * why
    union = area1 + area2 - inter
    iou = inter / (union + 1e-8)
    ltx2, lty2 = jnp.minimum(bx0, tx0), jnp.minimum(by0, ty0)
    rbx2, rby2 = jnp.maximum(bx1, tx1), jnp.maximum(by1, ty1)
    whx2, why2 = jnp.maximum(rbx2 - ltx2, 0.0), jnp.maximum(rby2 - lty2, 0.0)
    area = whx2 * why2
    giou = iou - (area - union) / (area + 1e-8)
    cost_giou = -giou

    c_ref[0] = (_COST_BBOX * cost_bbox + _COST_CLASS * cost_class) \
        + _COST_GIOU * cost_giou


def _assign_body(c_ref, rows_ref, cols_ref):
    big = jnp.float32(_BIG)
    Cm0 = jnp.concatenate(
        [c_ref[0], jnp.full((_Q, 128 - _T), big, jnp.float32)], axis=1)
    riota = lax.broadcasted_iota(jnp.int32, (_Q, 128), 0)
    ciota = lax.broadcasted_iota(jnp.int32, (_Q, 128), 1)
    flatbase = riota * _T + ciota
    lane = lax.broadcasted_iota(jnp.int32, (1, 128), 1)
    bigi = jnp.int32(2**30)

    def body(i, state):
        Cm, rowvec = state
        m = jnp.min(Cm)
        fi = jnp.min(jnp.where(Cm == m, flatbase, bigi))
        q = fi // _T
        t = fi - q * _T
        Cm = jnp.where((riota == q) | (ciota == t), big, Cm)
        rowvec = jnp.where(lane == t, q, rowvec)
        return Cm, rowvec

    _, rowvec = lax.fori_loop(
        0, _T, body, (Cm0, jnp.zeros((1, 128), jnp.int32)))
    rows_ref[0] = rowvec
    cols_ref[0] = lane


def kernel(pred_logits, pred_boxes, tgt_boxes, positive_map):
    tboxT = jnp.transpose(tgt_boxes, (0, 2, 1))
    pmapT = jnp.transpose(positive_map.astype(jnp.int32), (0, 2, 1))

    C = pl.pallas_call(
        _cost_body,
        grid=(_BS,),
        in_specs=[
            pl.BlockSpec((1, _Q, _C), lambda b: (b, 0, 0)),
            pl.BlockSpec((1, _Q, 4), lambda b: (b, 0, 0)),
            pl.BlockSpec((1, 4, _T), lambda b: (b, 0, 0)),
            pl.BlockSpec((1, 4, _T), lambda b: (b, 0, 0)),
        ],
        out_specs=pl.BlockSpec((1, _Q, _T), lambda b: (b, 0, 0)),
        out_shape=jax.ShapeDtypeStruct((_BS, _Q, _T), jnp.float32),
    )(pred_logits, pred_boxes, tboxT, pmapT)

    rows_p, cols_p = pl.pallas_call(
        _assign_body,
        grid=(_BS,),
        in_specs=[pl.BlockSpec((1, _Q, _T), lambda b: (b, 0, 0))],
        out_specs=[
            pl.BlockSpec((1, 1, 128), lambda b: (b, 0, 0)),
            pl.BlockSpec((1, 1, 128), lambda b: (b, 0, 0)),
        ],
        out_shape=[
            jax.ShapeDtypeStruct((_BS, 1, 128), jnp.int32),
            jax.ShapeDtypeStruct((_BS, 1, 128), jnp.int32),
        ],
    )(C)

    rows = rows_p[:, 0, :_T]
    cols = cols_p[:, 0, :_T]
    return (C, rows, cols)


# speedup vs baseline: 14.7508x; 3.7088x over previous
import functools

import jax
import jax.numpy as jnp
from jax import lax
from jax.experimental import pallas as pl
from jax.experimental.pallas import tpu as pltpu
from jax.experimental.pallas import tpu_sc as plsc

_COST_CLASS = 2.0
_COST_BBOX = 5.0
_COST_GIOU = 2.0
_ALPHA = 0.25

_BS, _Q, _C, _T, _P = 8, 900, 256, 100, 4
_QP = 912
_RMP = 1024
_TP = 128
_NCH = _RMP // 16
_BIGF = 1e30


def _cost_body(logits_ref, boxes_ref, tboxT_ref, pmapT_ref, c_ref, cp_ref):
    logits = logits_ref[0]
    p = jax.nn.sigmoid(logits)
    one_m_p = 1.0 - p
    neg = (1.0 - _ALPHA) * (p * p) * (-jnp.log(one_m_p))
    pos = _ALPHA * (one_m_p * one_m_p) * (-jnp.log(p + 1e-8))
    diff = pos - neg

    cio = lax.broadcasted_iota(jnp.int32, (_C, _T), 0)
    gs = []
    for pp in range(_P):
        idx = pmapT_ref[0, pp : pp + 1, :]
        onehot = (cio == idx).astype(jnp.float32)
        gs.append(lax.dot_general(
            diff, onehot, (((1,), (0,)), ((), ())),
            preferred_element_type=jnp.float32,
            precision=lax.Precision.HIGHEST,
        ))
    cost_class = ((gs[0] + gs[1]) + (gs[2] + gs[3])) * (1.0 / _P)

    boxes = boxes_ref[0]
    bcx, bcy, bw, bh = (boxes[:, k : k + 1] for k in range(4))
    tb = tboxT_ref[0]
    tcx, tcy, tw, th = (tb[k : k + 1, :] for k in range(4))

    cost_bbox = ((jnp.abs(bcx - tcx) + jnp.abs(bcy - tcy))
                 + jnp.abs(bw - tw)) + jnp.abs(bh - th)

    bx0, by0 = bcx - 0.5 * bw, bcy - 0.5 * bh
    bx1, by1 = bcx + 0.5 * bw, bcy + 0.5 * bh
    tx0, ty0 = tcx - 0.5 * tw, tcy - 0.5 * th
    tx1, ty1 = tcx + 0.5 * tw, tcy + 0.5 * th

    area1 = (bx1 - bx0) * (by1 - by0)
    area2 = (tx1 - tx0) * (ty1 - ty0)
    ltx, lty = jnp.maximum(bx0, tx0), jnp.maximum(by0, ty0)
    rbx, rby = jnp.minimum(bx1, tx1), jnp.minimum(by1, ty1)
    whx, why = jnp.maximum(rbx - ltx, 0.0), jnp.maximum(rby - lty, 0.0)
    inter = whx * why
    union = area1 + area2 - inter
    iou = inter / (union + 1e-8)
    ltx2, lty2 = jnp.minimum(bx0, tx0), jnp.minimum(by0, ty0)
    rbx2, rby2 = jnp.maximum(bx1, tx1), jnp.maximum(by1, ty1)
    whx2, why2 = jnp.maximum(rbx2 - ltx2, 0.0), jnp.maximum(rby2 - lty2, 0.0)
    area = whx2 * why2
    giou = iou - (area - union) / (area + 1e-8)
    cost_giou = -giou

    cost = (_COST_BBOX * cost_bbox + _COST_CLASS * cost_class) \
        + _COST_GIOU * cost_giou
    c_ref[0] = cost
    cp_ref[0, 0:_Q, 0:_T] = cost
    cp_ref[0, 0:_Q, _T:_TP] = jnp.full((_Q, _TP - _T), _BIGF, jnp.float32)
    cp_ref[0, _Q:_QP, :] = jnp.full((_QP - _Q, _TP), _BIGF, jnp.float32)


def _sc_assign(cp_hbm, rows_hbm, cols_hbm, cp_v, rm_v, cm_v, colp_v,
               rows_v, cols_v):
    c = lax.axis_index("c")
    s = lax.axis_index("s")
    wid = s * 2 + c
    iota = lax.broadcasted_iota(jnp.int32, (16,), 0)
    bigf_v = jnp.full((16,), _BIGF, jnp.float32)

    @pl.when(wid < _BS)
    def _work():
        b = wid
        pltpu.sync_copy(cp_hbm.at[b], cp_v)

        for j in range(_TP // 16):
            colp_v[pl.ds(16 * j, 16)] = jnp.zeros((16,), jnp.float32)
        for j in range(8):
            rows_v[pl.ds(16 * j, 16)] = jnp.zeros((16,), jnp.int32)
            cols_v[pl.ds(16 * j, 16)] = 16 * j + iota
        for j in range(_RMP // 16):
            rm_v[pl.ds(16 * j, 16)] = bigf_v

        def row_slice(r, j):
            off = pl.multiple_of(r * _TP + 16 * j, 16)
            return cp_v[pl.ds(off, 16)]

        def row_min(r):
            rv = row_slice(r, 0)
            for j in range(1, _TP // 16):
                rv = jnp.minimum(rv, row_slice(r, j))
            return jnp.min(rv)

        def init_chunk(jc, _):
            acc = jnp.full((16,), _BIGF, jnp.float32)
            for l in range(16):
                acc = jnp.where(iota == l, row_min(16 * jc + l), acc)
            off = pl.multiple_of(16 * jc, 16)
            rm_v[pl.ds(off, 16)] = acc
            return 0
        lax.fori_loop(0, _QP // 16, init_chunk, 0)

        for jj in range(_NCH // 16):
            acc = jnp.full((16,), _BIGF, jnp.float32)
            for l in range(16):
                v = jnp.min(rm_v[pl.ds(16 * (16 * jj + l), 16)])
                acc = jnp.where(iota == l, v, acc)
            cm_v[pl.ds(16 * jj, 16)] = acc

        def update_cm(jc):
            off = pl.multiple_of(16 * jc, 16)
            nv = jnp.min(rm_v[pl.ds(off, 16)])
            coff = pl.multiple_of(16 * (jc // 16), 16)
            cch = cm_v[pl.ds(coff, 16)]
            cm_v[pl.ds(coff, 16)] = jnp.where(iota == jc % 16, nv, cch)

        def pick_body(carry):
            chs = [cm_v[pl.ds(16 * jj, 16)] for jj in range(_NCH // 16)]
            acc = jnp.minimum(jnp.minimum(chs[0], chs[1]),
                              jnp.minimum(chs[2], chs[3]))
            m = jnp.min(acc)
            jcand = jnp.full((16,), _NCH, jnp.int32)
            for jj, ch in enumerate(chs):
                jcand = jnp.minimum(
                    jcand, jnp.where(ch == m, 16 * jj + iota, _NCH))
            jc = jnp.min(jcand)
            roff = pl.multiple_of(16 * jc, 16)
            rch = rm_v[pl.ds(roff, 16)]
            lane = jnp.min(jnp.where(rch == m, iota, 16))
            q = 16 * jc + lane
            bestv = jnp.full((16,), _BIGF, jnp.float32)
            bestc = jnp.full((16,), _TP, jnp.int32)
            for j in range(_TP // 16):
                cidx = 16 * j + iota
                v = row_slice(q, j)
                pz = colp_v[pl.ds(16 * j, 16)]
                v = jnp.where(pz > 0.0, _BIGF, v)
                upd = v < bestv
                bestv = jnp.where(upd, v, bestv)
                bestc = jnp.where(upd, cidx, bestc)
            rv = jnp.min(bestv)
            t = jnp.min(jnp.where(bestv == rv, bestc, _TP))
            rm_v[pl.ds(roff, 16)] = jnp.where(iota == lane, rv, rch)
            update_cm(jc)
            return (rv == m, q, t)

        def iter_body(i, _):
            init = (jnp.bool_(False), jnp.int32(0), jnp.int32(0))
            _, q, t = lax.while_loop(
                lambda cr: jnp.logical_not(cr[0]), pick_body, init)
            toff = pl.multiple_of(16 * (t // 16), 16)
            tlane = t % 16
            rwch = rows_v[pl.ds(toff, 16)]
            rows_v[pl.ds(toff, 16)] = jnp.where(iota == tlane, q, rwch)
            pch = colp_v[pl.ds(toff, 16)]
            colp_v[pl.ds(toff, 16)] = jnp.where(iota == tlane, _BIGF, pch)
            jc = q // 16
            qoff = pl.multiple_of(16 * jc, 16)
            rch = rm_v[pl.ds(qoff, 16)]
            rm_v[pl.ds(qoff, 16)] = jnp.where(iota == q % 16, _BIGF, rch)
            update_cm(jc)
            return 0
        lax.fori_loop(0, _T, iter_body, 0)

        pltpu.sync_copy(rows_v, rows_hbm.at[b])
        pltpu.sync_copy(cols_v, cols_hbm.at[b])


def kernel(pred_logits, pred_boxes, tgt_boxes, positive_map):
    tboxT = jnp.transpose(tgt_boxes, (0, 2, 1))
    pmapT = jnp.transpose(positive_map.astype(jnp.int32), (0, 2, 1))

    C, Cp = pl.pallas_call(
        _cost_body,
        grid=(_BS,),
        in_specs=[
            pl.BlockSpec((1, _Q, _C), lambda b: (b, 0, 0)),
            pl.BlockSpec((1, _Q, 4), lambda b: (b, 0, 0)),
            pl.BlockSpec((1, 4, _T), lambda b: (b, 0, 0)),
            pl.BlockSpec((1, 4, _T), lambda b: (b, 0, 0)),
        ],
        out_specs=[
            pl.BlockSpec((1, _Q, _T), lambda b: (b, 0, 0)),
            pl.BlockSpec((1, _QP, _TP), lambda b: (b, 0, 0)),
        ],
        out_shape=[
            jax.ShapeDtypeStruct((_BS, _Q, _T), jnp.float32),
            jax.ShapeDtypeStruct((_BS, _QP, _TP), jnp.float32),
        ],
    )(pred_logits, pred_boxes, tboxT, pmapT)

    mesh = plsc.VectorSubcoreMesh(core_axis_name="c", subcore_axis_name="s")
    rows_p, cols_p = pl.kernel(
        _sc_assign,
        out_type=[
            jax.ShapeDtypeStruct((_BS, 128), jnp.int32),
            jax.ShapeDtypeStruct((_BS, 128), jnp.int32),
        ],
        mesh=mesh,
        compiler_params=pltpu.CompilerParams(needs_layout_passes=False),
        scratch_types=[
            pltpu.VMEM((_QP * _TP,), jnp.float32),
            pltpu.VMEM((_RMP,), jnp.float32),
            pltpu.VMEM((_NCH,), jnp.float32),
            pltpu.VMEM((_TP,), jnp.float32),
            pltpu.VMEM((128,), jnp.int32),
            pltpu.VMEM((128,), jnp.int32),
        ],
    )(jnp.reshape(Cp, (_BS, _QP * _TP)))

    rows = rows_p[:, :_T]
    cols = cols_p[:, :_T]
    return (C, rows, cols)


# speedup vs baseline: 15.1399x; 1.0264x over previous
import functools

import jax
import jax.numpy as jnp
from jax import lax
from jax.experimental import pallas as pl
from jax.experimental.pallas import tpu as pltpu
from jax.experimental.pallas import tpu_sc as plsc

_COST_CLASS = 2.0
_COST_BBOX = 5.0
_COST_GIOU = 2.0
_ALPHA = 0.25

_BS, _Q, _C, _T, _P = 8, 900, 256, 100, 4
_QP = 912
_RMP = 1024
_TP = 128
_NCH = _RMP // 16
_BIGF = 1e30


def _cost_body(logits_ref, boxes_ref, tboxT_ref, pmapT_ref, c_ref, cp_ref,
               rm_ref):
    logits = logits_ref[0]
    p = jax.nn.sigmoid(logits)
    one_m_p = 1.0 - p
    neg = (1.0 - _ALPHA) * (p * p) * (-jnp.log(one_m_p))
    pos = _ALPHA * (one_m_p * one_m_p) * (-jnp.log(p + 1e-8))
    diff = pos - neg

    cio = lax.broadcasted_iota(jnp.int32, (_C, _T), 0)
    gs = []
    for pp in range(_P):
        idx = pmapT_ref[0, pp : pp + 1, :]
        onehot = (cio == idx).astype(jnp.float32)
        gs.append(lax.dot_general(
            diff, onehot, (((1,), (0,)), ((), ())),
            preferred_element_type=jnp.float32,
            precision=lax.Precision.HIGHEST,
        ))
    cost_class = ((gs[0] + gs[1]) + (gs[2] + gs[3])) * (1.0 / _P)

    boxes = boxes_ref[0]
    bcx, bcy, bw, bh = (boxes[:, k : k + 1] for k in range(4))
    tb = tboxT_ref[0]
    tcx, tcy, tw, th = (tb[k : k + 1, :] for k in range(4))

    cost_bbox = ((jnp.abs(bcx - tcx) + jnp.abs(bcy - tcy))
                 + jnp.abs(bw - tw)) + jnp.abs(bh - th)

    bx0, by0 = bcx - 0.5 * bw, bcy - 0.5 * bh
    bx1, by1 = bcx + 0.5 * bw, bcy + 0.5 * bh
    tx0, ty0 = tcx - 0.5 * tw, tcy - 0.5 * th
    tx1, ty1 = tcx + 0.5 * tw, tcy + 0.5 * th

    area1 = (bx1 - bx0) * (by1 - by0)
    area2 = (tx1 - tx0) * (ty1 - ty0)
    ltx, lty = jnp.maximum(bx0, tx0), jnp.maximum(by0, ty0)
    rbx, rby = jnp.minimum(bx1, tx1), jnp.minimum(by1, ty1)
    whx, why = jnp.maximum(rbx - ltx, 0.0), jnp.maximum(rby - lty, 0.0)
    inter = whx * why
    union = area1 + area2 - inter
    iou = inter / (union + 1e-8)
    ltx2, lty2 = jnp.minimum(bx0, tx0), jnp.minimum(by0, ty0)
    rbx2, rby2 = jnp.maximum(bx1, tx1), jnp.maximum(by1, ty1)
    whx2, why2 = jnp.maximum(rbx2 - ltx2, 0.0), jnp.maximum(rby2 - lty2, 0.0)
    area = whx2 * why2
    giou = iou - (area - union) / (area + 1e-8)
    cost_giou = -giou

    cost = (_COST_BBOX * cost_bbox + _COST_CLASS * cost_class) \
        + _COST_GIOU * cost_giou
    c_ref[0] = cost
    cp_ref[0, 0:_Q, 0:_T] = cost
    cp_ref[0, 0:_Q, _T:_TP] = jnp.full((_Q, _TP - _T), _BIGF, jnp.float32)
    cp_ref[0, _Q:_QP, :] = jnp.full((_QP - _Q, _TP), _BIGF, jnp.float32)
    rm = jnp.min(cost, axis=1, keepdims=True)
    rmT = jnp.transpose(rm, (1, 0))
    rm_ref[0] = jnp.concatenate(
        [rmT, jnp.full((1, _RMP - _Q), _BIGF, jnp.float32)], axis=1)


def _sc_assign(cp_hbm, rm_hbm, rows_hbm, cols_hbm, cp_v, rm_v, cm_v, colp_v,
               rows_v, cols_v):
    c = lax.axis_index("c")
    s = lax.axis_index("s")
    wid = s * 2 + c
    iota = lax.broadcasted_iota(jnp.int32, (16,), 0)
    bigf_v = jnp.full((16,), _BIGF, jnp.float32)

    @pl.when(wid < _BS)
    def _work():
        b = wid
        pltpu.sync_copy(cp_hbm.at[b], cp_v)
        pltpu.sync_copy(rm_hbm.at[b, 0], rm_v)

        for j in range(_TP // 16):
            colp_v[pl.ds(16 * j, 16)] = jnp.zeros((16,), jnp.float32)
        for j in range(8):
            rows_v[pl.ds(16 * j, 16)] = jnp.zeros((16,), jnp.int32)
            cols_v[pl.ds(16 * j, 16)] = 16 * j + iota

        def row_slice(r, j):
            off = pl.multiple_of(r * _TP + 16 * j, 16)
            return cp_v[pl.ds(off, 16)]

        for jj in range(_NCH // 16):
            acc = jnp.full((16,), _BIGF, jnp.float32)
            for l in range(16):
                v = jnp.min(rm_v[pl.ds(16 * (16 * jj + l), 16)])
                acc = jnp.where(iota == l, v, acc)
            cm_v[pl.ds(16 * jj, 16)] = acc

        def update_cm(jc):
            off = pl.multiple_of(16 * jc, 16)
            nv = jnp.min(rm_v[pl.ds(off, 16)])
            coff = pl.multiple_of(16 * (jc // 16), 16)
            cch = cm_v[pl.ds(coff, 16)]
            cm_v[pl.ds(coff, 16)] = jnp.where(iota == jc % 16, nv, cch)

        def pick_body(carry):
            chs = [cm_v[pl.ds(16 * jj, 16)] for jj in range(_NCH // 16)]
            acc = jnp.minimum(jnp.minimum(chs[0], chs[1]),
                              jnp.minimum(chs[2], chs[3]))
            m = jnp.min(acc)
            jcand = jnp.full((16,), _NCH, jnp.int32)
            for jj, ch in enumerate(chs):
                jcand = jnp.minimum(
                    jcand, jnp.where(ch == m, 16 * jj + iota, _NCH))
            jc = jnp.min(jcand)
            roff = pl.multiple_of(16 * jc, 16)
            rch = rm_v[pl.ds(roff, 16)]
            lane = jnp.min(jnp.where(rch == m, iota, 16))
            q = 16 * jc + lane
            bestv = jnp.full((16,), _BIGF, jnp.float32)
            bestc = jnp.full((16,), _TP, jnp.int32)
            for j in range(_TP // 16):
                cidx = 16 * j + iota
                v = row_slice(q, j)
                pz = colp_v[pl.ds(16 * j, 16)]
                v = jnp.where(pz > 0.0, _BIGF, v)
                upd = v < bestv
                bestv = jnp.where(upd, v, bestv)
                bestc = jnp.where(upd, cidx, bestc)
            rv = jnp.min(bestv)
            t = jnp.min(jnp.where(bestv == rv, bestc, _TP))
            rm_v[pl.ds(roff, 16)] = jnp.where(iota == lane, rv, rch)
            update_cm(jc)
            return (rv == m, q, t)

        def iter_body(i, _):
            init = (jnp.bool_(False), jnp.int32(0), jnp.int32(0))
            _, q, t = lax.while_loop(
                lambda cr: jnp.logical_not(cr[0]), pick_body, init)
            toff = pl.multiple_of(16 * (t // 16), 16)
            tlane = t % 16
            rwch = rows_v[pl.ds(toff, 16)]
            rows_v[pl.ds(toff, 16)] = jnp.where(iota == tlane, q, rwch)
            pch = colp_v[pl.ds(toff, 16)]
            colp_v[pl.ds(toff, 16)] = jnp.where(iota == tlane, _BIGF, pch)
            jc = q // 16
            qoff = pl.multiple_of(16 * jc, 16)
            rch = rm_v[pl.ds(qoff, 16)]
            rm_v[pl.ds(qoff, 16)] = jnp.where(iota == q % 16, _BIGF, rch)
            update_cm(jc)
            return 0
        lax.fori_loop(0, _T, iter_body, 0)

        pltpu.sync_copy(rows_v, rows_hbm.at[b])
        pltpu.sync_copy(cols_v, cols_hbm.at[b])


def kernel(pred_logits, pred_boxes, tgt_boxes, positive_map):
    tboxT = jnp.transpose(tgt_boxes, (0, 2, 1))
    pmapT = jnp.transpose(positive_map.astype(jnp.int32), (0, 2, 1))

    C, Cp, Rm = pl.pallas_call(
        _cost_body,
        grid=(_BS,),
        in_specs=[
            pl.BlockSpec((1, _Q, _C), lambda b: (b, 0, 0)),
            pl.BlockSpec((1, _Q, 4), lambda b: (b, 0, 0)),
            pl.BlockSpec((1, 4, _T), lambda b: (b, 0, 0)),
            pl.BlockSpec((1, 4, _T), lambda b: (b, 0, 0)),
        ],
        out_specs=[
            pl.BlockSpec((1, _Q, _T), lambda b: (b, 0, 0)),
            pl.BlockSpec((1, _QP, _TP), lambda b: (b, 0, 0)),
            pl.BlockSpec((1, 1, _RMP), lambda b: (b, 0, 0)),
        ],
        out_shape=[
            jax.ShapeDtypeStruct((_BS, _Q, _T), jnp.float32),
            jax.ShapeDtypeStruct((_BS, _QP, _TP), jnp.float32),
            jax.ShapeDtypeStruct((_BS, 1, _RMP), jnp.float32),
        ],
    )(pred_logits, pred_boxes, tboxT, pmapT)

    mesh = plsc.VectorSubcoreMesh(core_axis_name="c", subcore_axis_name="s")
    rows_p, cols_p = pl.kernel(
        _sc_assign,
        out_type=[
            jax.ShapeDtypeStruct((_BS, 128), jnp.int32),
            jax.ShapeDtypeStruct((_BS, 128), jnp.int32),
        ],
        mesh=mesh,
        compiler_params=pltpu.CompilerParams(needs_layout_passes=False),
        scratch_types=[
            pltpu.VMEM((_QP * _TP,), jnp.float32),
            pltpu.VMEM((_RMP,), jnp.float32),
            pltpu.VMEM((_NCH,), jnp.float32),
            pltpu.VMEM((_TP,), jnp.float32),
            pltpu.VMEM((128,), jnp.int32),
            pltpu.VMEM((128,), jnp.int32),
        ],
    )(jnp.reshape(Cp, (_BS, _QP * _TP)), Rm)

    rows = rows_p[:, :_T]
    cols = cols_p[:, :_T]
    return (C, rows, cols)


# speedup vs baseline: 15.1447x; 1.0003x over previous
import jax
import jax.numpy as jnp
from jax import lax
from jax.experimental import pallas as pl
from jax.experimental.pallas import tpu as pltpu
from jax.experimental.pallas import tpu_sc as plsc

_COST_CLASS = 2.0
_COST_BBOX = 5.0
_COST_GIOU = 2.0
_ALPHA = 0.25

_BS, _Q, _C, _T, _P = 8, 900, 256, 100, 4
_QP = 912
_RMP = 1024
_TP = 128
_NCH = _RMP // 16
_BIGF = 1e30


def _cost_body(logits_ref, boxes_ref, tboxT_ref, pmapT_ref, c_ref, cp_ref,
               rm_ref):
    logits = logits_ref[0]
    p = jax.nn.sigmoid(logits)
    one_m_p = 1.0 - p
    neg = (1.0 - _ALPHA) * (p * p) * (-jnp.log(one_m_p))
    pos = _ALPHA * (one_m_p * one_m_p) * (-jnp.log(p + 1e-8))
    diff = pos - neg

    cio = lax.broadcasted_iota(jnp.int32, (_C, _T), 0)
    gs = []
    for pp in range(_P):
        idx = pmapT_ref[0, pp : pp + 1, :]
        onehot = (cio == idx).astype(jnp.float32)
        gs.append(lax.dot_general(
            diff, onehot, (((1,), (0,)), ((), ())),
            preferred_element_type=jnp.float32,
            precision=lax.Precision.HIGHEST,
        ))
    cost_class = ((gs[0] + gs[1]) + (gs[2] + gs[3])) * (1.0 / _P)

    boxes = boxes_ref[0]
    bcx, bcy, bw, bh = (boxes[:, k : k + 1] for k in range(4))
    tb = tboxT_ref[0]
    tcx, tcy, tw, th = (tb[k : k + 1, :] for k in range(4))

    cost_bbox = ((jnp.abs(bcx - tcx) + jnp.abs(bcy - tcy))
                 + jnp.abs(bw - tw)) + jnp.abs(bh - th)

    bx0, by0 = bcx - 0.5 * bw, bcy - 0.5 * bh
    bx1, by1 = bcx + 0.5 * bw, bcy + 0.5 * bh
    tx0, ty0 = tcx - 0.5 * tw, tcy - 0.5 * th
    tx1, ty1 = tcx + 0.5 * tw, tcy + 0.5 * th

    area1 = (bx1 - bx0) * (by1 - by0)
    area2 = (tx1 - tx0) * (ty1 - ty0)
    ltx, lty = jnp.maximum(bx0, tx0), jnp.maximum(by0, ty0)
    rbx, rby = jnp.minimum(bx1, tx1), jnp.minimum(by1, ty1)
    whx, why = jnp.maximum(rbx - ltx, 0.0), jnp.maximum(rby - lty, 0.0)
    inter = whx * why
    union = area1 + area2 - inter
    iou = inter / (union + 1e-8)
    ltx2, lty2 = jnp.minimum(bx0, tx0), jnp.minimum(by0, ty0)
    rbx2, rby2 = jnp.maximum(bx1, tx1), jnp.maximum(by1, ty1)
    whx2, why2 = jnp.maximum(rbx2 - ltx2, 0.0), jnp.maximum(rby2 - lty2, 0.0)
    area = whx2 * why2
    giou = iou - (area - union) / (area + 1e-8)
    cost_giou = -giou

    cost = (_COST_BBOX * cost_bbox + _COST_CLASS * cost_class) \
        + _COST_GIOU * cost_giou
    c_ref[0] = cost
    cp_ref[0, 0:_Q, 0:_T] = cost
    cp_ref[0, 0:_Q, _T:_TP] = jnp.full((_Q, _TP - _T), _BIGF, jnp.float32)
    cp_ref[0, _Q:_QP, :] = jnp.full((_QP - _Q, _TP), _BIGF, jnp.float32)
    rm = jnp.min(cost, axis=1, keepdims=True)
    rmT = jnp.transpose(rm, (1, 0))
    rm_ref[0] = jnp.concatenate(
        [rmT, jnp.full((1, _RMP - _Q), _BIGF, jnp.float32)], axis=1)


def _sc_assign(cp_hbm, rm_hbm, rows_hbm, cols_hbm, cp_v, rm_v, cm_v, colp_v,
               rows_v, cols_v):
    c = lax.axis_index("c")
    s = lax.axis_index("s")
    wid = s * 2 + c
    iota = lax.broadcasted_iota(jnp.int32, (16,), 0)
    bigf_v = jnp.full((16,), _BIGF, jnp.float32)

    @pl.when(wid < _BS)
    def _work():
        b = wid
        pltpu.sync_copy(cp_hbm.at[b], cp_v)
        pltpu.sync_copy(rm_hbm.at[b, 0], rm_v)

        for j in range(_TP // 16):
            colp_v[pl.ds(16 * j, 16)] = jnp.zeros((16,), jnp.float32)
        for j in range(8):
            rows_v[pl.ds(16 * j, 16)] = jnp.zeros((16,), jnp.int32)
            cols_v[pl.ds(16 * j, 16)] = 16 * j + iota

        def row_slice(r, j):
            off = pl.multiple_of(r * _TP + 16 * j, 16)
            return cp_v[pl.ds(off, 16)]

        for jj in range(_NCH // 16):
            acc = jnp.full((16,), _BIGF, jnp.float32)
            for l in range(16):
                v = jnp.min(rm_v[pl.ds(16 * (16 * jj + l), 16)])
                acc = jnp.where(iota == l, v, acc)
            cm_v[pl.ds(16 * jj, 16)] = acc

        def update_cm(jc):
            off = pl.multiple_of(16 * jc, 16)
            nv = jnp.min(rm_v[pl.ds(off, 16)])
            coff = pl.multiple_of(16 * (jc // 16), 16)
            cch = cm_v[pl.ds(coff, 16)]
            cm_v[pl.ds(coff, 16)] = jnp.where(iota == jc % 16, nv, cch)

        def pick_body(carry):
            chs = [cm_v[pl.ds(16 * jj, 16)] for jj in range(_NCH // 16)]
            acc = jnp.minimum(jnp.minimum(chs[0], chs[1]),
                              jnp.minimum(chs[2], chs[3]))
            m = jnp.min(acc)
            jcand = jnp.full((16,), _NCH, jnp.int32)
            for jj, ch in enumerate(chs):
                jcand = jnp.minimum(
                    jcand, jnp.where(ch == m, 16 * jj + iota, _NCH))
            jc = jnp.min(jcand)
            roff = pl.multiple_of(16 * jc, 16)
            rch = rm_v[pl.ds(roff, 16)]
            lane = jnp.min(jnp.where(rch == m, iota, 16))
            q = 16 * jc + lane
            bestv = jnp.full((16,), _BIGF, jnp.float32)
            bestc = jnp.full((16,), _TP, jnp.int32)
            for j in range(_TP // 16):
                cidx = 16 * j + iota
                v = row_slice(q, j)
                pz = colp_v[pl.ds(16 * j, 16)]
                v = jnp.where(pz > 0.0, _BIGF, v)
                upd = v < bestv
                bestv = jnp.where(upd, v, bestv)
                bestc = jnp.where(upd, cidx, bestc)
            rv = jnp.min(bestv)
            t = jnp.min(jnp.where(bestv == rv, bestc, _TP))
            rm_v[pl.ds(roff, 16)] = jnp.where(iota == lane, rv, rch)
            update_cm(jc)
            return (rv == m, q, t)

        def iter_body(i, _):
            init = (jnp.bool_(False), jnp.int32(0), jnp.int32(0))
            _, q, t = lax.while_loop(
                lambda cr: jnp.logical_not(cr[0]), pick_body, init)
            toff = pl.multiple_of(16 * (t // 16), 16)
            tlane = t % 16
            rwch = rows_v[pl.ds(toff, 16)]
            rows_v[pl.ds(toff, 16)] = jnp.where(iota == tlane, q, rwch)
            pch = colp_v[pl.ds(toff, 16)]
            colp_v[pl.ds(toff, 16)] = jnp.where(iota == tlane, _BIGF, pch)
            jc = q // 16
            qoff = pl.multiple_of(16 * jc, 16)
            rch = rm_v[pl.ds(qoff, 16)]
            rm_v[pl.ds(qoff, 16)] = jnp.where(iota == q % 16, _BIGF, rch)
            update_cm(jc)
            return 0
        lax.fori_loop(0, _T, iter_body, 0)

        pltpu.sync_copy(rows_v, rows_hbm.at[b])
        pltpu.sync_copy(cols_v, cols_hbm.at[b])


def kernel(pred_logits, pred_boxes, tgt_boxes, positive_map):
    tboxT = jnp.transpose(tgt_boxes, (0, 2, 1))
    pmapT = jnp.transpose(positive_map.astype(jnp.int32), (0, 2, 1))

    C, Cp, Rm = pl.pallas_call(
        _cost_body,
        grid=(_BS,),
        in_specs=[
            pl.BlockSpec((1, _Q, _C), lambda b: (b, 0, 0)),
            pl.BlockSpec((1, _Q, 4), lambda b: (b, 0, 0)),
            pl.BlockSpec((1, 4, _T), lambda b: (b, 0, 0)),
            pl.BlockSpec((1, 4, _T), lambda b: (b, 0, 0)),
        ],
        out_specs=[
            pl.BlockSpec((1, _Q, _T), lambda b: (b, 0, 0)),
            pl.BlockSpec((1, _QP, _TP), lambda b: (b, 0, 0)),
            pl.BlockSpec((1, 1, _RMP), lambda b: (b, 0, 0)),
        ],
        out_shape=[
            jax.ShapeDtypeStruct((_BS, _Q, _T), jnp.float32),
            jax.ShapeDtypeStruct((_BS, _QP, _TP), jnp.float32),
            jax.ShapeDtypeStruct((_BS, 1, _RMP), jnp.float32),
        ],
    )(pred_logits, pred_boxes, tboxT, pmapT)

    mesh = plsc.VectorSubcoreMesh(core_axis_name="c", subcore_axis_name="s")
    rows_p, cols_p = pl.kernel(
        _sc_assign,
        out_type=[
            jax.ShapeDtypeStruct((_BS, 128), jnp.int32),
            jax.ShapeDtypeStruct((_BS, 128), jnp.int32),
        ],
        mesh=mesh,
        compiler_params=pltpu.CompilerParams(needs_layout_passes=False),
        scratch_types=[
            pltpu.VMEM((_QP * _TP,), jnp.float32),
            pltpu.VMEM((_RMP,), jnp.float32),
            pltpu.VMEM((_NCH,), jnp.float32),
            pltpu.VMEM((_TP,), jnp.float32),
            pltpu.VMEM((128,), jnp.int32),
            pltpu.VMEM((128,), jnp.int32),
        ],
    )(jnp.reshape(Cp, (_BS, _QP * _TP)), Rm)

    rows = rows_p[:, :_T]
    cols = cols_p[:, :_T]
    return (C, rows, cols)
